# baseline (speedup 1.0000x reference)
import functools

import jax
import jax.numpy as jnp
from jax import lax
from jax.experimental import pallas as pl
from jax.experimental.pallas import tpu as pltpu
from jax.experimental.pallas import tpu_sc as plsc

VOCAB = 1000002
EMBED_DIM = 32
N_TOKENS = 819200
N_BLOCKS = N_TOKENS // 128

_info = plsc.get_sparse_core_info()
_NW = _info.num_cores * _info.num_subcores
_B_PER_W = N_TOKENS // _NW
_CHUNK = 640
_BLK_PER_CHUNK = _CHUNK // 128
_N_CHUNKS = _B_PER_W // _CHUNK


def _embed_body(idx_hbm, table_hbm, out_hbm, idx_v, rows_v, stage_v,
                gsem0, gsem1, ssem0, ssem1):
    wid = lax.axis_index("s") * _info.num_cores + lax.axis_index("c")
    base = wid * _B_PER_W
    blk_base = wid * (_B_PER_W // 128)
    gsems = (gsem0, gsem1)
    ssems = (ssem0, ssem1)
    iota16 = lax.iota(jnp.int32, 16)

    def start_gather(g, b):
        idx_ref = idx_v.at[pl.ds(g * _CHUNK, _CHUNK)]
        pltpu.make_async_copy(table_hbm.at[idx_ref], rows_v.at[b],
                              gsems[b]).start()

    def wait_gather(b):
        pltpu.make_async_copy(table_hbm.at[pl.ds(0, _CHUNK)], rows_v.at[b],
                              gsems[b]).wait()

    def start_store(g, b):
        jb0 = blk_base + g * _BLK_PER_CHUNK
        pltpu.make_async_copy(stage_v.at[b],
                              out_hbm.at[:, pl.ds(jb0, _BLK_PER_CHUNK)],
                              ssems[b]).start()

    def wait_store(b):
        pltpu.make_async_copy(out_hbm.at[:, pl.ds(0, _BLK_PER_CHUNK)],
                              stage_v.at[b], ssems[b]).wait()

    def transpose(b):
        def per_block(jb, carry):
            for i in range(4):
                for s in range(8):
                    col = jnp.full((16,), 8 * i + s, dtype=jnp.int32)
                    for u16 in range(8):
                        row = jb * 128 + u16 * 16 + iota16
                        vals = plsc.load_gather(rows_v.at[b], [row, col])
                        stage_v[b, i, jb, s, pl.ds(u16 * 16, 16)] = vals
            return carry

        lax.fori_loop(0, _BLK_PER_CHUNK, per_block, 0)

    def step(g, b, first, last):
        wait_gather(b)
        if not last:
            start_gather(g + 1, 1 - b)
        if not first:
            wait_store(b)
        transpose(b)
        start_store(g, b)

    pltpu.sync_copy(idx_hbm.at[pl.ds(base, _B_PER_W)], idx_v)

    start_gather(0, 0)
    step(0, 0, first=True, last=False)
    step(1, 1, first=True, last=False)
    step(2, 0, first=False, last=False)

    def superstep(ss, carry):
        step(3 + 2 * ss, 1, first=False, last=False)
        step(4 + 2 * ss, 0, first=False, last=False)
        return carry

    lax.fori_loop(0, (_N_CHUNKS - 4) // 2, superstep, 0)

    step(_N_CHUNKS - 1, 1, first=False, last=True)
    wait_store(0)
    wait_store(1)


@jax.jit
def _embed(indices, table):
    mesh = plsc.VectorSubcoreMesh(core_axis_name="c", subcore_axis_name="s")
    f = functools.partial(
        pl.kernel,
        mesh=mesh,
        out_type=jax.ShapeDtypeStruct((4, N_BLOCKS, 8, 128), jnp.float32),
        scratch_types=[
            pltpu.VMEM((_B_PER_W,), jnp.int32),
            pltpu.VMEM((2, _CHUNK, EMBED_DIM), jnp.float32),
            pltpu.VMEM((2, 4, _BLK_PER_CHUNK, 8, 128), jnp.float32),
            pltpu.SemaphoreType.DMA,
            pltpu.SemaphoreType.DMA,
            pltpu.SemaphoreType.DMA,
            pltpu.SemaphoreType.DMA,
        ],
        compiler_params=pltpu.CompilerParams(use_tc_tiling_on_sc=False,
                                             needs_layout_passes=False),
    )(_embed_body)
    p = f(indices, table)
    return jnp.transpose(p, (1, 3, 0, 2)).reshape(N_TOKENS, EMBED_DIM)


def kernel(indices, table):
    return _embed(indices, table)


# speedup vs baseline: 1.1661x; 1.1661x over previous
import functools

import jax
import jax.numpy as jnp
from jax import lax
from jax.experimental import pallas as pl
from jax.experimental.pallas import tpu as pltpu
from jax.experimental.pallas import tpu_sc as plsc

VOCAB = 1000002
EMBED_DIM = 32
N_TOKENS = 819200
N_BLOCKS = N_TOKENS // 128

_info = plsc.get_sparse_core_info()
_NW = _info.num_cores * _info.num_subcores
_B_PER_W = N_TOKENS // _NW
_CHUNK = 640
_BLK_PER_CHUNK = _CHUNK // 128
_N_CHUNKS = _B_PER_W // _CHUNK


def _embed_body(idx_hbm, table_hbm, out_hbm, idx_v, rows_v, stage_v,
                gsem0, gsem1, ssem0, ssem1):
    wid = lax.axis_index("s") * _info.num_cores + lax.axis_index("c")
    base = wid * _B_PER_W
    blk_base = wid * (_B_PER_W // 128)
    gsems = (gsem0, gsem1)
    ssems = (ssem0, ssem1)
    iota16 = lax.iota(jnp.int32, 16)

    def start_gather(g, b):
        idx_ref = idx_v.at[pl.ds(g * _CHUNK, _CHUNK)]
        pltpu.make_async_copy(table_hbm.at[idx_ref], rows_v.at[b],
                              gsems[b]).start()

    def wait_gather(b):
        pltpu.make_async_copy(table_hbm.at[pl.ds(0, _CHUNK)], rows_v.at[b],
                              gsems[b]).wait()

    def start_store(g, b):
        jb0 = blk_base + g * _BLK_PER_CHUNK
        pltpu.make_async_copy(stage_v.at[b],
                              out_hbm.at[:, pl.ds(jb0, _BLK_PER_CHUNK)],
                              ssems[b]).start()

    def wait_store(b):
        pltpu.make_async_copy(out_hbm.at[:, pl.ds(0, _BLK_PER_CHUNK)],
                              stage_v.at[b], ssems[b]).wait()

    def transpose(b):
        @plsc.parallel_loop(0, _BLK_PER_CHUNK * 256, unroll=8)
        def body(k):
            u16 = k % 8
            s = (k // 8) % 8
            i = (k // 64) % 4
            jb = k // 256
            row = jb * 128 + u16 * 16 + iota16
            col = jnp.full((16,), 8 * i + s, dtype=jnp.int32)
            vals = plsc.load_gather(rows_v.at[b], [row, col])
            stage_v[b, i, jb, s, pl.ds(u16 * 16, 16)] = vals

    def step(g, b, first, last):
        wait_gather(b)
        if not last:
            start_gather(g + 1, 1 - b)
        if not first:
            wait_store(b)
        transpose(b)
        start_store(g, b)

    pltpu.sync_copy(idx_hbm.at[pl.ds(base, _B_PER_W)], idx_v)

    start_gather(0, 0)
    step(0, 0, first=True, last=False)
    step(1, 1, first=True, last=False)
    step(2, 0, first=False, last=False)

    def superstep(ss, carry):
        step(3 + 2 * ss, 1, first=False, last=False)
        step(4 + 2 * ss, 0, first=False, last=False)
        return carry

    lax.fori_loop(0, (_N_CHUNKS - 4) // 2, superstep, 0)

    step(_N_CHUNKS - 1, 1, first=False, last=True)
    wait_store(0)
    wait_store(1)


@jax.jit
def _embed(indices, table):
    mesh = plsc.VectorSubcoreMesh(core_axis_name="c", subcore_axis_name="s")
    f = functools.partial(
        pl.kernel,
        mesh=mesh,
        out_type=jax.ShapeDtypeStruct((4, N_BLOCKS, 8, 128), jnp.float32),
        scratch_types=[
            pltpu.VMEM((_B_PER_W,), jnp.int32),
            pltpu.VMEM((2, _CHUNK, EMBED_DIM), jnp.float32),
            pltpu.VMEM((2, 4, _BLK_PER_CHUNK, 8, 128), jnp.float32),
            pltpu.SemaphoreType.DMA,
            pltpu.SemaphoreType.DMA,
            pltpu.SemaphoreType.DMA,
            pltpu.SemaphoreType.DMA,
        ],
        compiler_params=pltpu.CompilerParams(use_tc_tiling_on_sc=False,
                                             needs_layout_passes=False),
    )(_embed_body)
    p = f(indices, table)
    return jnp.transpose(p, (1, 3, 0, 2)).reshape(N_TOKENS, EMBED_DIM)


def kernel(indices, table):
    return _embed(indices, table)


# speedup vs baseline: 1.3131x; 1.1261x over previous
import functools

import jax
import jax.numpy as jnp
from jax import lax
from jax.experimental import pallas as pl
from jax.experimental.pallas import tpu as pltpu
from jax.experimental.pallas import tpu_sc as plsc

VOCAB = 1000002
EMBED_DIM = 32
N_TOKENS = 819200
N_BLOCKS = N_TOKENS // 128

_info = plsc.get_sparse_core_info()
_NW = _info.num_cores * _info.num_subcores
_B_PER_W = N_TOKENS // _NW
_CHUNK = 640
_BLK_PER_CHUNK = _CHUNK // 128
_N_CHUNKS = _B_PER_W // _CHUNK
_STAGE = _CHUNK * EMBED_DIM
_ISTRIDE = N_BLOCKS * 1024


def _embed_body(idx_hbm, table_hbm, out_hbm, idx_v, rows_v, stage_v,
                gsem0, gsem1, ssem0, ssem1):
    wid = lax.axis_index("s") * _info.num_cores + lax.axis_index("c")
    base = wid * _B_PER_W
    blk_base = wid * (_B_PER_W // 128)
    gsems = (gsem0, gsem1)
    ssems = (ssem0, ssem1)
    iota16 = lax.iota(jnp.int32, 16)
    lanes = 5120 * (iota16 // 8) + 128 * (iota16 % 8)
    vh = (lanes, lanes + 10240)

    def start_gather(g, b):
        idx_ref = idx_v.at[pl.ds(g * _CHUNK, _CHUNK)]
        pltpu.make_async_copy(table_hbm.at[idx_ref], rows_v.at[b],
                              gsems[b]).start()

    def wait_gather(b):
        pltpu.make_async_copy(table_hbm.at[pl.ds(0, _CHUNK)], rows_v.at[b],
                              gsems[b]).wait()

    def start_store(g, b):
        jb0 = blk_base + g * _BLK_PER_CHUNK
        for i in range(4):
            pltpu.make_async_copy(
                stage_v.at[b, pl.ds(i * (_STAGE // 4), _STAGE // 4)],
                out_hbm.at[pl.ds(i * _ISTRIDE + jb0 * 1024, _STAGE // 4)],
                ssems[b]).start()

    def wait_store(b):
        pltpu.make_async_copy(out_hbm.at[pl.ds(0, _STAGE)], stage_v.at[b],
                              ssems[b]).wait()

    def transpose(b):
        @plsc.parallel_loop(0, _CHUNK, unroll=8)
        def body(t):
            sc = ((t >> 7) << 10) | (t & 127)
            for h in (0, 1):
                vals = rows_v[b, t, pl.ds(16 * h, 16)]
                plsc.store_scatter(stage_v.at[b], [vh[h] + sc], vals)

    def step(g, b, first, last):
        wait_gather(b)
        if not last:
            start_gather(g + 1, 1 - b)
        if not first:
            wait_store(b)
        transpose(b)
        start_store(g, b)

    pltpu.sync_copy(idx_hbm.at[pl.ds(base, _B_PER_W)], idx_v)

    start_gather(0, 0)
    step(0, 0, first=True, last=False)
    step(1, 1, first=True, last=False)
    step(2, 0, first=False, last=False)

    def superstep(ss, carry):
        step(3 + 2 * ss, 1, first=False, last=False)
        step(4 + 2 * ss, 0, first=False, last=False)
        return carry

    lax.fori_loop(0, (_N_CHUNKS - 4) // 2, superstep, 0)

    step(_N_CHUNKS - 1, 1, first=False, last=True)
    wait_store(0)
    wait_store(1)


@jax.jit
def _embed(indices, table):
    mesh = plsc.VectorSubcoreMesh(core_axis_name="c", subcore_axis_name="s")
    f = functools.partial(
        pl.kernel,
        mesh=mesh,
        out_type=jax.ShapeDtypeStruct((N_TOKENS * EMBED_DIM,), jnp.float32),
        scratch_types=[
            pltpu.VMEM((_B_PER_W,), jnp.int32),
            pltpu.VMEM((2, _CHUNK, EMBED_DIM), jnp.float32),
            pltpu.VMEM((2, _STAGE), jnp.float32),
            pltpu.SemaphoreType.DMA,
            pltpu.SemaphoreType.DMA,
            pltpu.SemaphoreType.DMA,
            pltpu.SemaphoreType.DMA,
        ],
        compiler_params=pltpu.CompilerParams(use_tc_tiling_on_sc=False,
                                             needs_layout_passes=False),
    )(_embed_body)
    p = f(indices, table).reshape(4, N_BLOCKS, 8, 128)
    return jnp.transpose(p, (1, 3, 0, 2)).reshape(N_TOKENS, EMBED_DIM)


def kernel(indices, table):
    return _embed(indices, table)
